# baseline (speedup 1.0000x reference)
import functools

import jax
import jax.numpy as jnp
from jax import lax
from jax.experimental import pallas as pl
from jax.experimental.pallas import tpu as pltpu
from jax.experimental.pallas import tpu_sc as plsc

NUM_LAYERS = 3
EMB = 128
EDGE_DIM = 16
N_NODES = 10000
N_EDGES = 320000

NC = 2
NS = 16
EPT = N_EDGES // (NC * NS)
CHUNK = 80
NCHUNKS = EPT // CHUNK
N_PAD = 10240
ROWS_PT = N_PAD // NS
RCHUNK = CHUNK
NRC = ROWS_PT // RCHUNK

_VEC = 16
_VPR = EMB // _VEC


def _bn_body(h_ref, g_ref, b_ref, o_ref):
    h = h_ref[...]
    mu = jnp.mean(h, axis=0, keepdims=True)
    var = jnp.mean((h - mu) ** 2, axis=0, keepdims=True)
    o_ref[...] = (h - mu) / jnp.sqrt(var + 1e-5) * g_ref[...] + b_ref[...]


def _bn(h, gamma, beta):
    return pl.pallas_call(
        _bn_body,
        out_shape=jax.ShapeDtypeStruct((N_NODES, EMB), jnp.float32),
    )(h, gamma.reshape(1, EMB), beta.reshape(1, EMB))


_BE = 2000


def _edge_embed_body(ea_ref, w_ref, o_ref):
    t = jnp.dot(ea_ref[...], w_ref[...], preferred_element_type=jnp.float32)
    u = jax.lax.bitcast_convert_type(
        t.astype(jnp.bfloat16), jnp.uint16).astype(jnp.uint32)
    v = u.reshape(_BE // 2, 2, EMB)
    o_ref[...] = v[:, 0, :] | (v[:, 1, :] << 16)


def _edge_embed(edge_attr, edge_W_l):
    return pl.pallas_call(
        _edge_embed_body,
        grid=(N_EDGES // _BE,),
        in_specs=[
            pl.BlockSpec((_BE, EDGE_DIM), lambda i: (i, 0)),
            pl.BlockSpec((EDGE_DIM, EMB), lambda i: (0, 0)),
        ],
        out_specs=pl.BlockSpec((_BE // 2, EMB), lambda i: (i, 0)),
        out_shape=jax.ShapeDtypeStruct((N_EDGES // 2, EMB), jnp.uint32),
    )(edge_attr, edge_W_l)


def _mlp_body(hb_ref, agg_ref, w1_ref, b1_ref, w2_ref, b2_ref, eps_ref,
              g_ref, be_ref, o_ref, *, last):
    t = ((1.0 + eps_ref[...]) * hb_ref[...]
         + agg_ref[0, :N_NODES] + agg_ref[1, :N_NODES])
    t = jnp.maximum(
        jnp.dot(t, w1_ref[...], preferred_element_type=jnp.float32) + b1_ref[...],
        0.0)
    t = jnp.dot(t, w2_ref[...], preferred_element_type=jnp.float32) + b2_ref[...]
    if not last:
        t = jnp.maximum(t, 0.0)
        mu = jnp.mean(t, axis=0, keepdims=True)
        var = jnp.mean((t - mu) ** 2, axis=0, keepdims=True)
        t = (t - mu) / jnp.sqrt(var + 1e-5) * g_ref[...] + be_ref[...]
    o_ref[...] = t


def _mlp(hb, agg, w1, b1, w2, b2, eps_l, gamma_n, beta_n, last):
    body = functools.partial(_mlp_body, last=last)
    return pl.pallas_call(
        body,
        out_shape=jax.ShapeDtypeStruct((N_NODES, EMB), jnp.float32),
    )(hb, agg, w1, b1.reshape(1, EMB), w2, b2.reshape(1, EMB),
      eps_l.reshape(1, 1), gamma_n.reshape(1, EMB), beta_n.reshape(1, EMB))


def _sc_agg_body(hb, srci, dsti, e2, out,
                 sidx, didx, hbufs, ebufs, agg, sem_i, sem_g, sem_e, sem_d):
    c = lax.axis_index("c")
    s = lax.axis_index("s")
    tile = c * NS + s

    zero16 = jnp.zeros((_VEC,), jnp.float32)

    def zrow(r, carry):
        for k in range(_VPR):
            hbufs[0, r, pl.ds(k * _VEC, _VEC)] = zero16
        return carry

    lax.fori_loop(0, CHUNK, zrow, 0)
    for j in range(NRC):
        pltpu.sync_copy(hbufs.at[0],
                        agg.at[pl.ds(s * ROWS_PT + j * RCHUNK, RCHUNK)])
    plsc.subcore_barrier()

    ebase = tile * EPT

    def _issue_idx(g, b):
        pltpu.async_copy(srci.at[pl.ds(ebase + g * CHUNK, CHUNK)],
                         sidx.at[b], sem_i[b])
        pltpu.async_copy(dsti.at[pl.ds(ebase + g * CHUNK, CHUNK)],
                         didx.at[b], sem_d[b])

    def _wait_idx(g, b):
        pltpu.make_async_copy(srci.at[pl.ds(ebase + g * CHUNK, CHUNK)],
                              sidx.at[b], sem_i[b]).wait()
        pltpu.make_async_copy(dsti.at[pl.ds(ebase + g * CHUNK, CHUNK)],
                              didx.at[b], sem_d[b]).wait()

    pbase = tile * (EPT // 2)

    def _issue_data(g, b):
        pltpu.async_copy(hb.at[sidx.at[b]], hbufs.at[b], sem_g[b])
        pltpu.async_copy(e2.at[pl.ds(pbase + g * (CHUNK // 2), CHUNK // 2)],
                         ebufs.at[b], sem_e[b])

    def _process(g, b):
        nb = 1 - b
        pltpu.make_async_copy(hb.at[sidx.at[b]], hbufs.at[b], sem_g[b]).wait()
        pltpu.make_async_copy(
            e2.at[pl.ds(pbase + g * (CHUNK // 2), CHUNK // 2)],
            ebufs.at[b], sem_e[b]).wait()

        @pl.when(g + 1 < NCHUNKS)
        def _():
            _wait_idx(g + 1, nb)
            _issue_data(g + 1, nb)

        def row(q, inner):
            for k in range(_VPR):
                sl = pl.ds(k * _VEC, _VEC)
                w = ebufs[b, q, sl]
                e_lo = jax.lax.bitcast_convert_type(w << 16, jnp.float32)
                e_hi = jax.lax.bitcast_convert_type((w >> 16) << 16,
                                                    jnp.float32)
                hbufs[b, 2 * q, sl] = jnp.maximum(
                    hbufs[b, 2 * q, sl] + e_lo, 0.0)
                hbufs[b, 2 * q + 1, sl] = jnp.maximum(
                    hbufs[b, 2 * q + 1, sl] + e_hi, 0.0)
            return inner

        lax.fori_loop(0, CHUNK // 2, row, 0)
        pltpu.sync_copy(hbufs.at[b], agg.at[didx.at[b]], add=True)

        @pl.when(g + 2 < NCHUNKS)
        def _():
            _issue_idx(g + 2, b)

    _issue_idx(0, 0)
    _issue_idx(1, 1)
    _wait_idx(0, 0)
    _issue_data(0, 0)

    def pair(p, carry):
        for b in range(2):
            g = 2 * p + b

            @pl.when(g < NCHUNKS)
            def _():
                _process(g, b)
        return carry

    lax.fori_loop(0, (NCHUNKS + 2) // 2, pair, 0)
    plsc.subcore_barrier()

    r0 = s * ROWS_PT
    pltpu.sync_copy(agg.at[pl.ds(r0, ROWS_PT)], out.at[c, pl.ds(r0, ROWS_PT)])


@functools.lru_cache(maxsize=None)
def _make_sc_agg():
    mesh = plsc.VectorSubcoreMesh(
        core_axis_name="c", subcore_axis_name="s",
        num_cores=NC, num_subcores=NS)
    return pl.kernel(
        _sc_agg_body,
        out_type=jax.ShapeDtypeStruct((NC, N_PAD, EMB), jnp.float32),
        mesh=mesh,
        scratch_types=[
            pltpu.VMEM((2, CHUNK), jnp.int32),
            pltpu.VMEM((2, CHUNK), jnp.int32),
            pltpu.VMEM((2, CHUNK, EMB), jnp.float32),
            pltpu.VMEM((2, CHUNK // 2, EMB), jnp.uint32),
            pltpu.VMEM_SHARED((N_PAD, EMB), jnp.float32),
            [pltpu.SemaphoreType.DMA] * 2,
            [pltpu.SemaphoreType.DMA] * 2,
            [pltpu.SemaphoreType.DMA] * 2,
            [pltpu.SemaphoreType.DMA] * 2,
        ],
    )


def kernel(x, edge_index, edge_attr, batch, bn_gamma, bn_beta, edge_W,
           W1, b1, W2, b2, eps):
    src = edge_index[0].astype(jnp.int32)
    dst = edge_index[1].astype(jnp.int32)
    es = [_edge_embed(edge_attr, edge_W[l]) for l in range(NUM_LAYERS)]

    hb = _bn(x, bn_gamma[0], bn_beta[0])
    for l in range(NUM_LAYERS):
        last = l == NUM_LAYERS - 1
        agg = _make_sc_agg()(hb, src, dst, es[l])
        gamma_n = bn_gamma[0 if last else l + 1]
        beta_n = bn_beta[0 if last else l + 1]
        hb = _mlp(hb, agg, W1[l], b1[l], W2[l], b2[l], eps[l],
                  gamma_n, beta_n, last)
    return hb


# speedup vs baseline: 1.4818x; 1.4818x over previous
import functools

import jax
import jax.numpy as jnp
from jax import lax
from jax.experimental import pallas as pl
from jax.experimental.pallas import tpu as pltpu
from jax.experimental.pallas import tpu_sc as plsc

NUM_LAYERS = 3
EMB = 128
EDGE_DIM = 16
N_NODES = 10000
N_EDGES = 320000

NC = 2
NS = 16
EPT = N_EDGES // (NC * NS)
CHUNK = 80
NCHUNKS = EPT // CHUNK
N_PAD = 10240
ROWS_PT = N_PAD // NS
RCHUNK = CHUNK
NRC = ROWS_PT // RCHUNK

_VEC = 16
_VPR = EMB // _VEC


def _bn_body(h_ref, g_ref, b_ref, o_ref):
    h = h_ref[...]
    mu = jnp.mean(h, axis=0, keepdims=True)
    var = jnp.mean((h - mu) ** 2, axis=0, keepdims=True)
    o_ref[...] = (h - mu) / jnp.sqrt(var + 1e-5) * g_ref[...] + b_ref[...]


def _bn(h, gamma, beta):
    return pl.pallas_call(
        _bn_body,
        out_shape=jax.ShapeDtypeStruct((N_NODES, EMB), jnp.float32),
    )(h, gamma.reshape(1, EMB), beta.reshape(1, EMB))


_BE = 2000


def _edge_embed_body(ea_ref, w_ref, o_ref):
    o_ref[...] = jnp.dot(ea_ref[...], w_ref[...],
                         preferred_element_type=jnp.float32)


def _edge_embed(edge_attr, edge_W_l):
    return pl.pallas_call(
        _edge_embed_body,
        grid=(N_EDGES // _BE,),
        in_specs=[
            pl.BlockSpec((_BE, EDGE_DIM), lambda i: (i, 0)),
            pl.BlockSpec((EDGE_DIM, EMB), lambda i: (0, 0)),
        ],
        out_specs=pl.BlockSpec((_BE, EMB), lambda i: (i, 0)),
        out_shape=jax.ShapeDtypeStruct((N_EDGES, EMB), jnp.float32),
    )(edge_attr, edge_W_l)


def _mlp_body(hb_ref, agg_ref, w1_ref, b1_ref, w2_ref, b2_ref, eps_ref,
              g_ref, be_ref, o_ref, *, last):
    t = ((1.0 + eps_ref[...]) * hb_ref[...]
         + agg_ref[0, :N_NODES] + agg_ref[1, :N_NODES])
    t = jnp.maximum(
        jnp.dot(t, w1_ref[...], preferred_element_type=jnp.float32) + b1_ref[...],
        0.0)
    t = jnp.dot(t, w2_ref[...], preferred_element_type=jnp.float32) + b2_ref[...]
    if not last:
        t = jnp.maximum(t, 0.0)
        mu = jnp.mean(t, axis=0, keepdims=True)
        var = jnp.mean((t - mu) ** 2, axis=0, keepdims=True)
        t = (t - mu) / jnp.sqrt(var + 1e-5) * g_ref[...] + be_ref[...]
    o_ref[...] = t


def _mlp(hb, agg, w1, b1, w2, b2, eps_l, gamma_n, beta_n, last):
    body = functools.partial(_mlp_body, last=last)
    return pl.pallas_call(
        body,
        out_shape=jax.ShapeDtypeStruct((N_NODES, EMB), jnp.float32),
    )(hb, agg, w1, b1.reshape(1, EMB), w2, b2.reshape(1, EMB),
      eps_l.reshape(1, 1), gamma_n.reshape(1, EMB), beta_n.reshape(1, EMB))


def _sc_agg_body(hb, srci, dsti, e2, out,
                 sidx, didx, hbufs, ebufs, agg,
                 sem_i, sem_g, sem_e, sem_d, sem_s):
    c = lax.axis_index("c")
    s = lax.axis_index("s")
    tile = c * NS + s

    zero16 = jnp.zeros((_VEC,), jnp.float32)

    def zrow(r, carry):
        for k in range(_VPR):
            hbufs[0, r, pl.ds(k * _VEC, _VEC)] = zero16
        return carry

    lax.fori_loop(0, CHUNK, zrow, 0)
    for j in range(NRC):
        pltpu.sync_copy(hbufs.at[0],
                        agg.at[pl.ds(s * ROWS_PT + j * RCHUNK, RCHUNK)])
    plsc.subcore_barrier()

    ebase = tile * EPT

    def _issue_sidx(g, b):
        pltpu.async_copy(srci.at[pl.ds(ebase + g * CHUNK, CHUNK)],
                         sidx.at[b], sem_i[b])

    def _issue_didx(g, b):
        pltpu.async_copy(dsti.at[pl.ds(ebase + g * CHUNK, CHUNK)],
                         didx.at[b], sem_d[b])

    def _wait_sidx(g, b):
        pltpu.make_async_copy(srci.at[pl.ds(ebase + g * CHUNK, CHUNK)],
                              sidx.at[b], sem_i[b]).wait()

    def _issue_data(g, b):
        pltpu.async_copy(hb.at[sidx.at[b]], hbufs.at[b], sem_g[b])
        pltpu.async_copy(e2.at[pl.ds(ebase + g * CHUNK, CHUNK)],
                         ebufs.at[b], sem_e[b])

    def _wait_scat(b):
        pltpu.make_async_copy(hbufs.at[b], agg.at[didx.at[b]],
                              sem_s[b]).wait()

    def _process(g, b, wait_s):
        nb = 1 - b
        pltpu.make_async_copy(hb.at[sidx.at[b]], hbufs.at[b], sem_g[b]).wait()
        pltpu.make_async_copy(e2.at[pl.ds(ebase + g * CHUNK, CHUNK)],
                              ebufs.at[b], sem_e[b]).wait()
        if wait_s:
            _wait_scat(nb)

        @pl.when(g + 1 < NCHUNKS)
        def _():
            _issue_didx(g + 1, nb)

        @pl.when(g + 2 < NCHUNKS)
        def _():
            _issue_sidx(g + 2, b)

        @pl.when(g + 1 < NCHUNKS)
        def _():
            _wait_sidx(g + 1, nb)
            _issue_data(g + 1, nb)

        def row(r, inner):
            for u in range(2):
                for k in range(_VPR):
                    sl = pl.ds(k * _VEC, _VEC)
                    hbufs[b, 2 * r + u, sl] = jnp.maximum(
                        hbufs[b, 2 * r + u, sl] + ebufs[b, 2 * r + u, sl], 0.0)
            return inner

        lax.fori_loop(0, CHUNK // 2, row, 0)
        pltpu.make_async_copy(dsti.at[pl.ds(ebase + g * CHUNK, CHUNK)],
                              didx.at[b], sem_d[b]).wait()
        pltpu.async_copy(hbufs.at[b], agg.at[didx.at[b]], sem_s[b], add=True)

    _issue_sidx(0, 0)
    _issue_sidx(1, 1)
    _issue_didx(0, 0)
    _wait_sidx(0, 0)
    _issue_data(0, 0)
    _process(0, 0, False)

    def pair(p, carry):
        for b in range(2):
            g = 2 * p + 1 + b

            @pl.when(g < NCHUNKS)
            def _():
                _process(g, 1 - b, True)
        return carry

    lax.fori_loop(0, NCHUNKS // 2, pair, 0)
    _wait_scat(0)
    plsc.subcore_barrier()

    r0 = s * ROWS_PT
    pltpu.sync_copy(agg.at[pl.ds(r0, ROWS_PT)], out.at[c, pl.ds(r0, ROWS_PT)])


@functools.lru_cache(maxsize=None)
def _make_sc_agg():
    mesh = plsc.VectorSubcoreMesh(
        core_axis_name="c", subcore_axis_name="s",
        num_cores=NC, num_subcores=NS)
    return pl.kernel(
        _sc_agg_body,
        out_type=jax.ShapeDtypeStruct((NC, N_PAD, EMB), jnp.float32),
        mesh=mesh,
        scratch_types=[
            pltpu.VMEM((2, CHUNK), jnp.int32),
            pltpu.VMEM((2, CHUNK), jnp.int32),
            pltpu.VMEM((2, CHUNK, EMB), jnp.float32),
            pltpu.VMEM((2, CHUNK, EMB), jnp.float32),
            pltpu.VMEM_SHARED((N_PAD, EMB), jnp.float32),
            [pltpu.SemaphoreType.DMA] * 2,
            [pltpu.SemaphoreType.DMA] * 2,
            [pltpu.SemaphoreType.DMA] * 2,
            [pltpu.SemaphoreType.DMA] * 2,
            [pltpu.SemaphoreType.DMA] * 2,
        ],
    )


def kernel(x, edge_index, edge_attr, batch, bn_gamma, bn_beta, edge_W,
           W1, b1, W2, b2, eps):
    src = edge_index[0].astype(jnp.int32)
    dst = edge_index[1].astype(jnp.int32)
    es = [_edge_embed(edge_attr, edge_W[l]) for l in range(NUM_LAYERS)]

    hb = _bn(x, bn_gamma[0], bn_beta[0])
    for l in range(NUM_LAYERS):
        last = l == NUM_LAYERS - 1
        agg = _make_sc_agg()(hb, src, dst, es[l])
        gamma_n = bn_gamma[0 if last else l + 1]
        beta_n = bn_beta[0 if last else l + 1]
        hb = _mlp(hb, agg, W1[l], b1[l], W2[l], b2[l], eps[l],
                  gamma_n, beta_n, last)
    return hb


# speedup vs baseline: 1.5037x; 1.0148x over previous
import functools

import jax
import jax.numpy as jnp
from jax import lax
from jax.experimental import pallas as pl
from jax.experimental.pallas import tpu as pltpu
from jax.experimental.pallas import tpu_sc as plsc

NUM_LAYERS = 3
EMB = 128
EDGE_DIM = 16
N_NODES = 10000
N_EDGES = 320000

NC = 2
NS = 16
EPT = N_EDGES // (NC * NS)
CHUNK = 80
NCHUNKS = EPT // CHUNK
N_PAD = 10240
ROWS_PT = N_PAD // NS
RCHUNK = CHUNK
NRC = ROWS_PT // RCHUNK

_VEC = 16
_VPR = EMB // _VEC


def _bn_body(h_ref, g_ref, b_ref, o_ref):
    h = h_ref[...]
    mu = jnp.mean(h, axis=0, keepdims=True)
    var = jnp.mean((h - mu) ** 2, axis=0, keepdims=True)
    o_ref[...] = (h - mu) / jnp.sqrt(var + 1e-5) * g_ref[...] + b_ref[...]


def _bn(h, gamma, beta):
    return pl.pallas_call(
        _bn_body,
        out_shape=jax.ShapeDtypeStruct((N_NODES, EMB), jnp.float32),
    )(h, gamma.reshape(1, EMB), beta.reshape(1, EMB))


_BE = 4000


def _edge_embed_body(ea_ref, w_ref, o_ref):
    o_ref[...] = jnp.dot(ea_ref[...], w_ref[...],
                         preferred_element_type=jnp.float32)


def _edge_embed(edge_attr, edge_W_l):
    return pl.pallas_call(
        _edge_embed_body,
        grid=(N_EDGES // _BE,),
        in_specs=[
            pl.BlockSpec((_BE, EDGE_DIM), lambda i: (i, 0)),
            pl.BlockSpec((EDGE_DIM, EMB), lambda i: (0, 0)),
        ],
        out_specs=pl.BlockSpec((_BE, EMB), lambda i: (i, 0)),
        out_shape=jax.ShapeDtypeStruct((N_EDGES, EMB), jnp.float32),
    )(edge_attr, edge_W_l)


def _mlp_body(hb_ref, agg_ref, w1_ref, b1_ref, w2_ref, b2_ref, eps_ref,
              g_ref, be_ref, o_ref, *, last):
    t = ((1.0 + eps_ref[...]) * hb_ref[...]
         + agg_ref[0, :N_NODES] + agg_ref[1, :N_NODES])
    t = jnp.maximum(
        jnp.dot(t, w1_ref[...], preferred_element_type=jnp.float32) + b1_ref[...],
        0.0)
    t = jnp.dot(t, w2_ref[...], preferred_element_type=jnp.float32) + b2_ref[...]
    if not last:
        t = jnp.maximum(t, 0.0)
        mu = jnp.mean(t, axis=0, keepdims=True)
        var = jnp.mean((t - mu) ** 2, axis=0, keepdims=True)
        t = (t - mu) / jnp.sqrt(var + 1e-5) * g_ref[...] + be_ref[...]
    o_ref[...] = t


def _mlp(hb, agg, w1, b1, w2, b2, eps_l, gamma_n, beta_n, last):
    body = functools.partial(_mlp_body, last=last)
    return pl.pallas_call(
        body,
        out_shape=jax.ShapeDtypeStruct((N_NODES, EMB), jnp.float32),
    )(hb, agg, w1, b1.reshape(1, EMB), w2, b2.reshape(1, EMB),
      eps_l.reshape(1, 1), gamma_n.reshape(1, EMB), beta_n.reshape(1, EMB))


def _sc_agg_body(hb, srci, dsti, e2, out,
                 sidx, didx, hbufs, ebufs, agg,
                 sem_i, sem_g, sem_e, sem_d, sem_s):
    c = lax.axis_index("c")
    s = lax.axis_index("s")
    tile = c * NS + s

    zero16 = jnp.zeros((_VEC,), jnp.float32)

    def zrow(r, carry):
        for k in range(_VPR):
            hbufs[0, r, pl.ds(k * _VEC, _VEC)] = zero16
        return carry

    lax.fori_loop(0, CHUNK, zrow, 0)
    for j in range(NRC):
        pltpu.sync_copy(hbufs.at[0],
                        agg.at[pl.ds(s * ROWS_PT + j * RCHUNK, RCHUNK)])
    plsc.subcore_barrier()

    ebase = tile * EPT

    def _issue_sidx(g, b):
        pltpu.async_copy(srci.at[pl.ds(ebase + g * CHUNK, CHUNK)],
                         sidx.at[b], sem_i[b])

    def _issue_didx(g, b):
        pltpu.async_copy(dsti.at[pl.ds(ebase + g * CHUNK, CHUNK)],
                         didx.at[b], sem_d[b])

    def _wait_sidx(g, b):
        pltpu.make_async_copy(srci.at[pl.ds(ebase + g * CHUNK, CHUNK)],
                              sidx.at[b], sem_i[b]).wait()

    def _issue_data(g, b):
        pltpu.async_copy(hb.at[sidx.at[b]], hbufs.at[b], sem_g[b])
        pltpu.async_copy(e2.at[pl.ds(ebase + g * CHUNK, CHUNK)],
                         ebufs.at[b], sem_e[b])

    def _wait_scat(b):
        pltpu.make_async_copy(hbufs.at[b], agg.at[didx.at[b]],
                              sem_s[b]).wait()

    def _process(g, b, wait_s):
        nb = 1 - b
        pltpu.make_async_copy(hb.at[sidx.at[b]], hbufs.at[b], sem_g[b]).wait()
        pltpu.make_async_copy(e2.at[pl.ds(ebase + g * CHUNK, CHUNK)],
                              ebufs.at[b], sem_e[b]).wait()
        if wait_s:
            _wait_scat(nb)

        @pl.when(g + 1 < NCHUNKS)
        def _():
            _issue_didx(g + 1, nb)

        @pl.when(g + 2 < NCHUNKS)
        def _():
            _issue_sidx(g + 2, b)

        @pl.when(g + 1 < NCHUNKS)
        def _():
            _wait_sidx(g + 1, nb)
            _issue_data(g + 1, nb)

        def row(r, inner):
            for u in range(2):
                for k in range(_VPR):
                    sl = pl.ds(k * _VEC, _VEC)
                    hbufs[b, 2 * r + u, sl] = jnp.maximum(
                        hbufs[b, 2 * r + u, sl] + ebufs[b, 2 * r + u, sl], 0.0)
            return inner

        lax.fori_loop(0, CHUNK // 2, row, 0)
        pltpu.make_async_copy(dsti.at[pl.ds(ebase + g * CHUNK, CHUNK)],
                              didx.at[b], sem_d[b]).wait()
        pltpu.async_copy(hbufs.at[b], agg.at[didx.at[b]], sem_s[b], add=True)

    _issue_sidx(0, 0)
    _issue_sidx(1, 1)
    _issue_didx(0, 0)
    _wait_sidx(0, 0)
    _issue_data(0, 0)
    _process(0, 0, False)

    def pair(p, carry):
        for b in range(2):
            g = 2 * p + 1 + b

            @pl.when(g < NCHUNKS)
            def _():
                _process(g, 1 - b, True)
        return carry

    lax.fori_loop(0, NCHUNKS // 2, pair, 0)
    _wait_scat(0)
    plsc.subcore_barrier()

    r0 = s * ROWS_PT
    pltpu.sync_copy(agg.at[pl.ds(r0, ROWS_PT)], out.at[c, pl.ds(r0, ROWS_PT)])


@functools.lru_cache(maxsize=None)
def _make_sc_agg():
    mesh = plsc.VectorSubcoreMesh(
        core_axis_name="c", subcore_axis_name="s",
        num_cores=NC, num_subcores=NS)
    return pl.kernel(
        _sc_agg_body,
        out_type=jax.ShapeDtypeStruct((NC, N_PAD, EMB), jnp.float32),
        mesh=mesh,
        scratch_types=[
            pltpu.VMEM((2, CHUNK), jnp.int32),
            pltpu.VMEM((2, CHUNK), jnp.int32),
            pltpu.VMEM((2, CHUNK, EMB), jnp.float32),
            pltpu.VMEM((2, CHUNK, EMB), jnp.float32),
            pltpu.VMEM_SHARED((N_PAD, EMB), jnp.float32),
            [pltpu.SemaphoreType.DMA] * 2,
            [pltpu.SemaphoreType.DMA] * 2,
            [pltpu.SemaphoreType.DMA] * 2,
            [pltpu.SemaphoreType.DMA] * 2,
            [pltpu.SemaphoreType.DMA] * 2,
        ],
    )


def kernel(x, edge_index, edge_attr, batch, bn_gamma, bn_beta, edge_W,
           W1, b1, W2, b2, eps):
    src = edge_index[0].astype(jnp.int32)
    dst = edge_index[1].astype(jnp.int32)
    es = [_edge_embed(edge_attr, edge_W[l]) for l in range(NUM_LAYERS)]

    hb = _bn(x, bn_gamma[0], bn_beta[0])
    for l in range(NUM_LAYERS):
        last = l == NUM_LAYERS - 1
        agg = _make_sc_agg()(hb, src, dst, es[l])
        gamma_n = bn_gamma[0 if last else l + 1]
        beta_n = bn_beta[0 if last else l + 1]
        hb = _mlp(hb, agg, W1[l], b1[l], W2[l], b2[l], eps[l],
                  gamma_n, beta_n, last)
    return hb


# speedup vs baseline: 1.5113x; 1.0050x over previous
import functools

import jax
import jax.numpy as jnp
from jax import lax
from jax.experimental import pallas as pl
from jax.experimental.pallas import tpu as pltpu
from jax.experimental.pallas import tpu_sc as plsc

NUM_LAYERS = 3
EMB = 128
EDGE_DIM = 16
N_NODES = 10000
N_EDGES = 320000

NC = 2
NS = 16
EPT = N_EDGES // (NC * NS)
CHUNK = 80
NCHUNKS = EPT // CHUNK
N_PAD = 10240
ROWS_PT = N_PAD // NS
RCHUNK = CHUNK
NRC = ROWS_PT // RCHUNK

_VEC = 16
_VPR = EMB // _VEC


def _bn_body(h_ref, g_ref, b_ref, o_ref):
    h = h_ref[...]
    mu = jnp.mean(h, axis=0, keepdims=True)
    var = jnp.mean((h - mu) ** 2, axis=0, keepdims=True)
    o_ref[...] = (h - mu) / jnp.sqrt(var + 1e-5) * g_ref[...] + b_ref[...]


def _bn(h, gamma, beta):
    return pl.pallas_call(
        _bn_body,
        out_shape=jax.ShapeDtypeStruct((N_NODES, EMB), jnp.float32),
    )(h, gamma.reshape(1, EMB), beta.reshape(1, EMB))


_BE = 8000


def _edge_embed_body(ea_ref, w_ref, o_ref):
    o_ref[...] = jnp.dot(ea_ref[...], w_ref[...],
                         preferred_element_type=jnp.float32)


def _edge_embed(edge_attr, edge_W_l):
    return pl.pallas_call(
        _edge_embed_body,
        grid=(N_EDGES // _BE,),
        in_specs=[
            pl.BlockSpec((_BE, EDGE_DIM), lambda i: (i, 0)),
            pl.BlockSpec((EDGE_DIM, EMB), lambda i: (0, 0)),
        ],
        out_specs=pl.BlockSpec((_BE, EMB), lambda i: (i, 0)),
        out_shape=jax.ShapeDtypeStruct((N_EDGES, EMB), jnp.float32),
    )(edge_attr, edge_W_l)


def _mlp_body(hb_ref, agg_ref, w1_ref, b1_ref, w2_ref, b2_ref, eps_ref,
              g_ref, be_ref, o_ref, *, last):
    t = ((1.0 + eps_ref[...]) * hb_ref[...]
         + agg_ref[0, :N_NODES] + agg_ref[1, :N_NODES])
    t = jnp.maximum(
        jnp.dot(t, w1_ref[...], preferred_element_type=jnp.float32) + b1_ref[...],
        0.0)
    t = jnp.dot(t, w2_ref[...], preferred_element_type=jnp.float32) + b2_ref[...]
    if not last:
        t = jnp.maximum(t, 0.0)
        mu = jnp.mean(t, axis=0, keepdims=True)
        var = jnp.mean((t - mu) ** 2, axis=0, keepdims=True)
        t = (t - mu) / jnp.sqrt(var + 1e-5) * g_ref[...] + be_ref[...]
    o_ref[...] = t


def _mlp(hb, agg, w1, b1, w2, b2, eps_l, gamma_n, beta_n, last):
    body = functools.partial(_mlp_body, last=last)
    return pl.pallas_call(
        body,
        out_shape=jax.ShapeDtypeStruct((N_NODES, EMB), jnp.float32),
    )(hb, agg, w1, b1.reshape(1, EMB), w2, b2.reshape(1, EMB),
      eps_l.reshape(1, 1), gamma_n.reshape(1, EMB), beta_n.reshape(1, EMB))


def _sc_agg_body(hb, srci, dsti, e2, out,
                 sidx, didx, hbufs, ebufs, agg,
                 sem_i, sem_g, sem_e, sem_d, sem_s):
    c = lax.axis_index("c")
    s = lax.axis_index("s")
    tile = c * NS + s

    zero16 = jnp.zeros((_VEC,), jnp.float32)

    def zrow(r, carry):
        for k in range(_VPR):
            hbufs[0, r, pl.ds(k * _VEC, _VEC)] = zero16
        return carry

    lax.fori_loop(0, CHUNK, zrow, 0)
    for j in range(NRC):
        pltpu.sync_copy(hbufs.at[0],
                        agg.at[pl.ds(s * ROWS_PT + j * RCHUNK, RCHUNK)])
    plsc.subcore_barrier()

    ebase = tile * EPT

    def _issue_sidx(g, b):
        pltpu.async_copy(srci.at[pl.ds(ebase + g * CHUNK, CHUNK)],
                         sidx.at[b], sem_i[b])

    def _issue_didx(g, b):
        pltpu.async_copy(dsti.at[pl.ds(ebase + g * CHUNK, CHUNK)],
                         didx.at[b], sem_d[b])

    def _wait_sidx(g, b):
        pltpu.make_async_copy(srci.at[pl.ds(ebase + g * CHUNK, CHUNK)],
                              sidx.at[b], sem_i[b]).wait()

    def _issue_data(g, b):
        pltpu.async_copy(hb.at[sidx.at[b]], hbufs.at[b], sem_g[b])
        pltpu.async_copy(e2.at[pl.ds(ebase + g * CHUNK, CHUNK)],
                         ebufs.at[b], sem_e[b])

    def _wait_scat(b):
        pltpu.make_async_copy(hbufs.at[b], agg.at[didx.at[b]],
                              sem_s[b]).wait()

    def _process(g, b, wait_s):
        nb = 1 - b
        pltpu.make_async_copy(hb.at[sidx.at[b]], hbufs.at[b], sem_g[b]).wait()
        pltpu.make_async_copy(e2.at[pl.ds(ebase + g * CHUNK, CHUNK)],
                              ebufs.at[b], sem_e[b]).wait()
        if wait_s:
            _wait_scat(nb)

        @pl.when(g + 1 < NCHUNKS)
        def _():
            _issue_didx(g + 1, nb)

        @pl.when(g + 2 < NCHUNKS)
        def _():
            _issue_sidx(g + 2, b)

        @pl.when(g + 1 < NCHUNKS)
        def _():
            _wait_sidx(g + 1, nb)
            _issue_data(g + 1, nb)

        def row(r, inner):
            for u in range(2):
                for k in range(_VPR):
                    sl = pl.ds(k * _VEC, _VEC)
                    hbufs[b, 2 * r + u, sl] = jnp.maximum(
                        hbufs[b, 2 * r + u, sl] + ebufs[b, 2 * r + u, sl], 0.0)
            return inner

        lax.fori_loop(0, CHUNK // 2, row, 0)
        pltpu.make_async_copy(dsti.at[pl.ds(ebase + g * CHUNK, CHUNK)],
                              didx.at[b], sem_d[b]).wait()
        pltpu.async_copy(hbufs.at[b], agg.at[didx.at[b]], sem_s[b], add=True)

    _issue_sidx(0, 0)
    _issue_sidx(1, 1)
    _issue_didx(0, 0)
    _wait_sidx(0, 0)
    _issue_data(0, 0)
    _process(0, 0, False)

    def pair(p, carry):
        for b in range(2):
            g = 2 * p + 1 + b

            @pl.when(g < NCHUNKS)
            def _():
                _process(g, 1 - b, True)
        return carry

    lax.fori_loop(0, NCHUNKS // 2, pair, 0)
    _wait_scat(0)
    plsc.subcore_barrier()

    r0 = s * ROWS_PT
    pltpu.sync_copy(agg.at[pl.ds(r0, ROWS_PT)], out.at[c, pl.ds(r0, ROWS_PT)])


@functools.lru_cache(maxsize=None)
def _make_sc_agg():
    mesh = plsc.VectorSubcoreMesh(
        core_axis_name="c", subcore_axis_name="s",
        num_cores=NC, num_subcores=NS)
    return pl.kernel(
        _sc_agg_body,
        out_type=jax.ShapeDtypeStruct((NC, N_PAD, EMB), jnp.float32),
        mesh=mesh,
        scratch_types=[
            pltpu.VMEM((2, CHUNK), jnp.int32),
            pltpu.VMEM((2, CHUNK), jnp.int32),
            pltpu.VMEM((2, CHUNK, EMB), jnp.float32),
            pltpu.VMEM((2, CHUNK, EMB), jnp.float32),
            pltpu.VMEM_SHARED((N_PAD, EMB), jnp.float32),
            [pltpu.SemaphoreType.DMA] * 2,
            [pltpu.SemaphoreType.DMA] * 2,
            [pltpu.SemaphoreType.DMA] * 2,
            [pltpu.SemaphoreType.DMA] * 2,
            [pltpu.SemaphoreType.DMA] * 2,
        ],
    )


def kernel(x, edge_index, edge_attr, batch, bn_gamma, bn_beta, edge_W,
           W1, b1, W2, b2, eps):
    src = edge_index[0].astype(jnp.int32)
    dst = edge_index[1].astype(jnp.int32)
    es = [_edge_embed(edge_attr, edge_W[l]) for l in range(NUM_LAYERS)]

    hb = _bn(x, bn_gamma[0], bn_beta[0])
    for l in range(NUM_LAYERS):
        last = l == NUM_LAYERS - 1
        agg = _make_sc_agg()(hb, src, dst, es[l])
        gamma_n = bn_gamma[0 if last else l + 1]
        beta_n = bn_beta[0 if last else l + 1]
        hb = _mlp(hb, agg, W1[l], b1[l], W2[l], b2[l], eps[l],
                  gamma_n, beta_n, last)
    return hb
